# baseline (speedup 1.0000x reference)
import jax
import jax.numpy as jnp
from jax import lax
from jax.experimental import pallas as pl
from jax.experimental.pallas import tpu as pltpu
from jax.experimental.pallas import tpu_sc as plsc

ROWS = 128
COLS = 32768
LANES = 16
NSLICES = COLS // LANES
CHUNK = 128
NCHUNKS = COLS // CHUNK
SLICES_PER_CHUNK = CHUNK // LANES
LCAP = 48
NUM_CORES = 2
NUM_SUBCORES = 16
NWORKERS = NUM_CORES * NUM_SUBCORES
ROWS_PER_W = ROWS // NWORKERS
MARGIN = 3e-3
UNROLL = 8

_GATHER_DNUMS = lax.GatherDimensionNumbers(
    offset_dims=(), collapsed_slice_dims=(0,), start_index_map=(0,)
)


def _perm(v, idx):
    return lax.gather(
        v,
        idx[:, None],
        _GATHER_DNUMS,
        slice_sizes=(1,),
        mode=lax.GatherScatterMode.PROMISE_IN_BOUNDS,
    )


def _mk_helpers():
    lane = lax.iota(jnp.int32, LANES)
    bfly = [jnp.bitwise_xor(lane, sh) for sh in (1, 2, 4, 8)]

    def allmax(v):
        for idx in bfly:
            v = jnp.maximum(v, _perm(v, idx))
        return v

    def allsum(v):
        for idx in bfly:
            v = v + _perm(v, idx)
        return v

    return lane, allmax, allsum


_ONES = lambda: jnp.full((LANES,), 1.0, jnp.float32)
_ZERO = lambda: jnp.zeros((LANES,), jnp.float32)


def _body_a(x_hbm, flags_hbm, stats_hbm, row_v, flag_v, stats_v):
    cid = lax.axis_index("c")
    sid = lax.axis_index("s")
    wid = sid * NUM_CORES + cid

    lane, allmax, allsum = _mk_helpers()
    ones_v, zero_v = _ONES(), _ZERO()

    def do_row(j, carry):
        r = wid * ROWS_PER_W + j
        pltpu.sync_copy(x_hbm.at[r], row_v)

        def maxbody(i, acc):
            for cu in range(2):
                g = i * 2 + cu
                base = g * CHUNK
                vs = [row_v[pl.ds(base + u * LANES, LANES)] for u in range(SLICES_PER_CHUNK)]
                m0 = jnp.maximum(vs[0], vs[1])
                m1 = jnp.maximum(vs[2], vs[3])
                m2 = jnp.maximum(vs[4], vs[5])
                m3 = jnp.maximum(vs[6], vs[7])
                mx = jnp.maximum(jnp.maximum(m0, m1), jnp.maximum(m2, m3))
                flag_v[pl.ds(g * LANES, LANES)] = mx
                acc = jnp.maximum(acc, mx)
            return acc

        acc = lax.fori_loop(0, NCHUNKS // 2, maxbody, row_v[pl.ds(0, LANES)])
        row_max = allmax(acc)

        def ks_at(t):
            def body(i, c):
                accs = list(c)
                base = i * (LANES * UNROLL)
                for u in range(UNROLL):
                    v = row_v[pl.ds(base + u * LANES, LANES)]
                    m = v > t
                    j = u % 4
                    accs[j] = accs[j] + jnp.where(m, ones_v, zero_v)
                    accs[4 + j] = accs[4 + j] + jnp.where(m, v, zero_v)
                return tuple(accs)

            accs = lax.fori_loop(0, NSLICES // UNROLL, body, (zero_v,) * 8)
            ka = (accs[0] + accs[1]) + (accs[2] + accs[3])
            sa = (accs[4] + accs[5]) + (accs[6] + accs[7])
            return allsum(ka), allsum(sa)

        lo = row_max - 1.001
        for _ in range(2):
            k, s = ks_at(lo)
            lo = (s - 1.0) / k
        lom = lo - MARGIN

        stats_v[pl.ds(0, LANES)] = lom
        stats_v[pl.ds(LANES, LANES)] = row_max
        pltpu.sync_copy(flag_v, flags_hbm.at[r])
        pltpu.sync_copy(stats_v, stats_hbm.at[r])
        return carry

    lax.fori_loop(0, ROWS_PER_W, do_row, 0)


def _body_b(
    x2_hbm, idx_hbm, stats_hbm, out2_hbm,
    rowbuf0, rowbuf1, cand4_v, idx4_v, stats4_v,
    sem_g, sem_i0, sem_i1, sem_o0, sem_o1,
):
    cid = lax.axis_index("c")
    sid = lax.axis_index("s")
    wid = sid * NUM_CORES + cid
    r0 = wid * ROWS_PER_W

    lane, allmax, allsum = _mk_helpers()
    ones_v, zero_v = _ONES(), _ZERO()
    NR = ROWS_PER_W
    CU = 4

    rowbuf = [rowbuf0, rowbuf1]
    sem_in = [sem_i0, sem_i1]
    sem_out = [sem_o0, sem_o1]

    d_in = [None] * NR
    for j in range(2):
        d_in[j] = pltpu.async_copy(
            x2_hbm.at[pl.ds((r0 + j) * NCHUNKS, NCHUNKS)], rowbuf[j], sem_in[j]
        )

    pltpu.sync_copy(idx_hbm.at[pl.ds(r0, NR)], idx4_v)
    pltpu.sync_copy(stats_hbm.at[pl.ds(r0, NR)], stats4_v)
    d_g = [
        pltpu.async_copy(x2_hbm.at[idx4_v.at[j]], cand4_v.at[j], sem_g)
        for j in range(NR)
    ]
    for d in d_g:
        d.wait()

    lo = [stats4_v[j, pl.ds(0, LANES)] for j in range(NR)]
    row_max = [stats4_v[j, pl.ds(LANES, LANES)] for j in range(NR)]

    def ks_at4(ts):
        def body(i, c):
            accs = list(c)
            for cu in range(CU):
                ci = i * CU + cu
                for u in range(SLICES_PER_CHUNK):
                    for j in range(NR):
                        v = cand4_v[j, ci, pl.ds(u * LANES, LANES)]
                        m = v > ts[j]
                        accs[j] = accs[j] + jnp.where(m, ones_v, zero_v)
                        accs[NR + j] = accs[NR + j] + jnp.where(m, v, zero_v)
            return tuple(accs)

        accs = lax.fori_loop(0, LCAP // CU, body, (zero_v,) * (2 * NR))
        return [(allsum(accs[j]), allsum(accs[NR + j])) for j in range(NR)]

    def fsum4(ts):
        def body(i, c):
            accs = list(c)
            for cu in range(CU):
                ci = i * CU + cu
                for u in range(SLICES_PER_CHUNK):
                    for j in range(NR):
                        v = cand4_v[j, ci, pl.ds(u * LANES, LANES)]
                        accs[j] = accs[j] + jnp.maximum(v - ts[j], 0.0)
            return tuple(accs)

        accs = lax.fori_loop(0, LCAP // CU, body, (zero_v,) * NR)
        return [allsum(accs[j]) for j in range(NR)]

    for _ in range(2):
        ks = ks_at4(lo)
        lo = [(s - 1.0) / k for k, s in ks]

    f_lo = fsum4(lo)
    hi = [
        jnp.maximum(jnp.minimum(lo[j] + (f_lo[j] - 1.0), row_max[j]), lo[j])
        for j in range(NR)
    ]

    def bis(i, c):
        blo = list(c[:NR])
        bhi = list(c[NR:])
        mid = [0.5 * (blo[j] + bhi[j]) for j in range(NR)]
        fs = fsum4(mid)
        for j in range(NR):
            gt = fs[j] > 1.0
            blo[j] = jnp.where(gt, mid[j], blo[j])
            bhi[j] = jnp.where(gt, bhi[j], mid[j])
        return tuple(blo) + tuple(bhi)

    res = lax.fori_loop(0, 16, bis, tuple(lo) + tuple(hi))
    lo = list(res[:NR])

    ks = ks_at4(lo)
    tau = [(s - 1.0) / k for k, s in ks]

    d_out = [None] * NR
    for j in range(NR):
        b = j % 2
        d_in[j].wait()
        buf = rowbuf[b]
        tj = tau[j]

        def outbody(i, c):
            for cu in range(2):
                for u in range(SLICES_PER_CHUNK):
                    sl = (i * 2 + cu, pl.ds(u * LANES, LANES))
                    buf[sl] = jnp.maximum(buf[sl] - tj, 0.0)
            return c

        lax.fori_loop(0, NCHUNKS // 2, outbody, 0)
        d_out[j] = pltpu.async_copy(
            buf, out2_hbm.at[pl.ds((r0 + j) * NCHUNKS, NCHUNKS)], sem_out[b]
        )
        if j + 2 < NR:
            d_out[j].wait()
            d_in[j + 2] = pltpu.async_copy(
                x2_hbm.at[pl.ds((r0 + j + 2) * NCHUNKS, NCHUNKS)],
                rowbuf[b],
                sem_in[b],
            )
    d_out[NR - 2].wait()
    d_out[NR - 1].wait()


def _body_c(x_hbm, out_hbm, row_v):
    cid = lax.axis_index("c")
    sid = lax.axis_index("s")
    wid = sid * NUM_CORES + cid

    lane, allmax, allsum = _mk_helpers()
    ones_v, zero_v = _ONES(), _ZERO()

    def do_row(j, carry):
        r = wid * ROWS_PER_W + j
        pltpu.sync_copy(x_hbm.at[r], row_v)

        def maxbody(i, acc):
            base = i * (LANES * UNROLL)
            for u in range(UNROLL):
                acc = jnp.maximum(acc, row_v[pl.ds(base + u * LANES, LANES)])
            return acc

        acc = lax.fori_loop(0, NSLICES // UNROLL, maxbody, row_v[pl.ds(0, LANES)])
        row_max = allmax(acc)

        def ks_at(t):
            def body(i, c):
                accs = list(c)
                base = i * (LANES * UNROLL)
                for u in range(UNROLL):
                    v = row_v[pl.ds(base + u * LANES, LANES)]
                    m = v > t
                    j = u % 4
                    accs[j] = accs[j] + jnp.where(m, ones_v, zero_v)
                    accs[4 + j] = accs[4 + j] + jnp.where(m, v, zero_v)
                return tuple(accs)

            accs = lax.fori_loop(0, NSLICES // UNROLL, body, (zero_v,) * 8)
            ka = (accs[0] + accs[1]) + (accs[2] + accs[3])
            sa = (accs[4] + accs[5]) + (accs[6] + accs[7])
            return allsum(ka), allsum(sa)

        def fsum(tau):
            def body(i, c):
                accs = list(c)
                base = i * (LANES * UNROLL)
                for u in range(UNROLL):
                    v = row_v[pl.ds(base + u * LANES, LANES)]
                    accs[u % 4] = accs[u % 4] + jnp.maximum(v - tau, 0.0)
                return tuple(accs)

            accs = lax.fori_loop(0, NSLICES // UNROLL, body, (zero_v,) * 4)
            return allsum((accs[0] + accs[1]) + (accs[2] + accs[3]))

        lo = row_max - 1.001
        for _ in range(4):
            k, s = ks_at(lo)
            lo = (s - 1.0) / k

        f_lo = fsum(lo)
        hi = jnp.minimum(lo + (f_lo - 1.0), row_max)
        hi = jnp.maximum(hi, lo)

        def bis(i, c):
            blo, bhi = c
            mid = 0.5 * (blo + bhi)
            gt = fsum(mid) > 1.0
            return (jnp.where(gt, mid, blo), jnp.where(gt, bhi, mid))

        lo, _ = lax.fori_loop(0, 26, bis, (lo, hi))

        k, s = ks_at(lo)
        tau = (s - 1.0) / k

        def outbody(i, c):
            base = i * (LANES * UNROLL)
            for u in range(UNROLL):
                sl = pl.ds(base + u * LANES, LANES)
                row_v[sl] = jnp.maximum(row_v[sl] - tau, 0.0)
            return c

        lax.fori_loop(0, NSLICES // UNROLL, outbody, 0)
        pltpu.sync_copy(row_v, out_hbm.at[r])
        return carry

    lax.fori_loop(0, ROWS_PER_W, do_row, 0)


def _mesh():
    return plsc.VectorSubcoreMesh(core_axis_name="c", subcore_axis_name="s")


def _kernel_a(x):
    fn = pl.kernel(
        _body_a,
        out_type=(
            jax.ShapeDtypeStruct((ROWS, NCHUNKS * LANES), jnp.float32),
            jax.ShapeDtypeStruct((ROWS, 2 * LANES), jnp.float32),
        ),
        mesh=_mesh(),
        scratch_types=[
            pltpu.VMEM((COLS,), jnp.float32),
            pltpu.VMEM((NCHUNKS * LANES,), jnp.float32),
            pltpu.VMEM((2 * LANES,), jnp.float32),
        ],
    )
    return fn(x)


def _kernel_b(x2, idx, stats):
    fn = pl.kernel(
        _body_b,
        out_type=jax.ShapeDtypeStruct((ROWS * NCHUNKS, CHUNK), jnp.float32),
        mesh=_mesh(),
        scratch_types=[
            pltpu.VMEM((NCHUNKS, CHUNK), jnp.float32),
            pltpu.VMEM((NCHUNKS, CHUNK), jnp.float32),
            pltpu.VMEM((ROWS_PER_W, LCAP, CHUNK), jnp.float32),
            pltpu.VMEM((ROWS_PER_W, LCAP), jnp.int32),
            pltpu.VMEM((ROWS_PER_W, 2 * LANES), jnp.float32),
            pltpu.SemaphoreType.DMA,
            pltpu.SemaphoreType.DMA,
            pltpu.SemaphoreType.DMA,
            pltpu.SemaphoreType.DMA,
            pltpu.SemaphoreType.DMA,
        ],
    )
    return fn(x2, idx, stats)


def _kernel_c(x):
    fn = pl.kernel(
        _body_c,
        out_type=jax.ShapeDtypeStruct((ROWS, COLS), jnp.float32),
        mesh=_mesh(),
        scratch_types=[pltpu.VMEM((COLS,), jnp.float32)],
    )
    return fn(x)


_STAGE = 2


@jax.jit
def _pipeline(x):
    flags, stats = _kernel_a(x)
    score = flags.reshape(ROWS, NCHUNKS, LANES).max(axis=2)
    lom = stats[:, 0:1]
    flag = score > lom
    cnt = jnp.sum(flag.astype(jnp.int32), axis=1)
    overflow = jnp.any(cnt > LCAP)

    _, top_idx = lax.top_k(score, LCAP)
    base = (jnp.arange(ROWS, dtype=jnp.int32) * NCHUNKS)[:, None]
    idx = base + top_idx.astype(jnp.int32)

    x2 = x.reshape(ROWS * NCHUNKS, CHUNK)

    def fast(ops):
        xx2, iidx, sstats = ops
        out2 = _kernel_b(xx2, iidx, sstats)
        return out2.reshape(ROWS, COLS)

    def slow(ops):
        xx2, _, _ = ops
        return _kernel_c(xx2.reshape(ROWS, COLS))

    return lax.cond(overflow, slow, fast, (x2, idx, stats)) if _STAGE == 2 else ((flags, stats) if _STAGE == 0 else (flags, stats, idx, overflow))


def kernel(input):
    return _pipeline(input)


# speedup vs baseline: 1.5599x; 1.5599x over previous
import jax
import jax.numpy as jnp
from jax import lax
from jax.experimental import pallas as pl
from jax.experimental.pallas import tpu as pltpu
from jax.experimental.pallas import tpu_sc as plsc

ROWS = 128
COLS = 32768
LANES = 16
NSLICES = COLS // LANES
CHUNK = 128
NCHUNKS = COLS // CHUNK
SLICES_PER_CHUNK = CHUNK // LANES
LCAP = 48
NUM_CORES = 2
NUM_SUBCORES = 16
NWORKERS = NUM_CORES * NUM_SUBCORES
ROWS_PER_W = ROWS // NWORKERS
MARGIN = 3e-3
UNROLL = 8

_GATHER_DNUMS = lax.GatherDimensionNumbers(
    offset_dims=(), collapsed_slice_dims=(0,), start_index_map=(0,)
)


def _perm(v, idx):
    return lax.gather(
        v,
        idx[:, None],
        _GATHER_DNUMS,
        slice_sizes=(1,),
        mode=lax.GatherScatterMode.PROMISE_IN_BOUNDS,
    )


def _mk_helpers():
    lane = lax.iota(jnp.int32, LANES)
    bfly = [jnp.bitwise_xor(lane, sh) for sh in (1, 2, 4, 8)]

    def allmax(v):
        for idx in bfly:
            v = jnp.maximum(v, _perm(v, idx))
        return v

    def allsum(v):
        for idx in bfly:
            v = v + _perm(v, idx)
        return v

    return lane, allmax, allsum


_ONES = lambda: jnp.full((LANES,), 1.0, jnp.float32)
_ZERO = lambda: jnp.zeros((LANES,), jnp.float32)


def _body_a(x_hbm, flags_hbm, stats_hbm, row_v, flag_v, stats_v):
    cid = lax.axis_index("c")
    sid = lax.axis_index("s")
    wid = sid * NUM_CORES + cid

    lane, allmax, allsum = _mk_helpers()
    ones_v, zero_v = _ONES(), _ZERO()

    def do_row(j, carry):
        r = wid * ROWS_PER_W + j
        pltpu.sync_copy(x_hbm.at[r], row_v)

        def maxbody(i, acc):
            for cu in range(2):
                g = i * 2 + cu
                base = g * CHUNK
                vs = [row_v[pl.ds(base + u * LANES, LANES)] for u in range(SLICES_PER_CHUNK)]
                m0 = jnp.maximum(vs[0], vs[1])
                m1 = jnp.maximum(vs[2], vs[3])
                m2 = jnp.maximum(vs[4], vs[5])
                m3 = jnp.maximum(vs[6], vs[7])
                mx = jnp.maximum(jnp.maximum(m0, m1), jnp.maximum(m2, m3))
                flag_v[pl.ds(g * LANES, LANES)] = mx
                acc = jnp.maximum(acc, mx)
            return acc

        acc = lax.fori_loop(0, NCHUNKS // 2, maxbody, row_v[pl.ds(0, LANES)])
        row_max = allmax(acc)

        def ks_at(t):
            def body(i, c):
                accs = list(c)
                base = i * (LANES * UNROLL)
                for u in range(UNROLL):
                    v = row_v[pl.ds(base + u * LANES, LANES)]
                    m = v > t
                    j = u % 4
                    accs[j] = accs[j] + jnp.where(m, ones_v, zero_v)
                    accs[4 + j] = accs[4 + j] + jnp.where(m, v, zero_v)
                return tuple(accs)

            accs = lax.fori_loop(0, NSLICES // UNROLL, body, (zero_v,) * 8)
            ka = (accs[0] + accs[1]) + (accs[2] + accs[3])
            sa = (accs[4] + accs[5]) + (accs[6] + accs[7])
            return allsum(ka), allsum(sa)

        lo = row_max - 1.001
        for _ in range(2):
            k, s = ks_at(lo)
            lo = (s - 1.0) / k
        lom = lo - MARGIN

        stats_v[pl.ds(0, LANES)] = lom
        stats_v[pl.ds(LANES, LANES)] = row_max
        pltpu.sync_copy(flag_v, flags_hbm.at[r])
        pltpu.sync_copy(stats_v, stats_hbm.at[r])
        return carry

    lax.fori_loop(0, ROWS_PER_W, do_row, 0)


def _body_b(
    x2_hbm, idx_hbm, stats_hbm, out2_hbm,
    rowbuf0, rowbuf1, cand_a, cand_b, cand_c, cand_d, idx4_v, stats4_v,
    sem_g, sem_i0, sem_i1, sem_o0, sem_o1,
):
    cid = lax.axis_index("c")
    sid = lax.axis_index("s")
    wid = sid * NUM_CORES + cid
    r0 = wid * ROWS_PER_W

    lane, allmax, allsum = _mk_helpers()
    ones_v, zero_v = _ONES(), _ZERO()
    NR = ROWS_PER_W
    CU = 2
    cands = [cand_a, cand_b, cand_c, cand_d]

    rowbuf = [rowbuf0, rowbuf1]
    sem_in = [sem_i0, sem_i1]
    sem_out = [sem_o0, sem_o1]

    d_in = [None] * NR
    for j in range(2):
        d_in[j] = pltpu.async_copy(
            x2_hbm.at[pl.ds((r0 + j) * NCHUNKS, NCHUNKS)], rowbuf[j], sem_in[j]
        )

    pltpu.sync_copy(idx_hbm.at[pl.ds(r0, NR)], idx4_v)
    pltpu.sync_copy(stats_hbm.at[pl.ds(r0, NR)], stats4_v)
    d_g = [
        pltpu.async_copy(x2_hbm.at[idx4_v.at[j]], cands[j], sem_g)
        for j in range(NR)
    ]
    for d in d_g:
        d.wait()

    lo = [stats4_v[j, pl.ds(0, LANES)] for j in range(NR)]
    row_max = [stats4_v[j, pl.ds(LANES, LANES)] for j in range(NR)]

    def ks_at4(ts):
        def body(i, c):
            accs = list(c)
            for cu in range(CU):
                ci = i * CU + cu
                for u in range(SLICES_PER_CHUNK):
                    for j in range(NR):
                        v = cands[j][ci, pl.ds(u * LANES, LANES)]
                        m = v > ts[j]
                        accs[j] = accs[j] + jnp.where(m, ones_v, zero_v)
                        accs[NR + j] = accs[NR + j] + jnp.where(m, v, zero_v)
            return tuple(accs)

        accs = lax.fori_loop(0, LCAP // CU, body, (zero_v,) * (2 * NR))
        return [(allsum(accs[j]), allsum(accs[NR + j])) for j in range(NR)]

    def fsum4(ts):
        def body(i, c):
            accs = list(c)
            for cu in range(CU):
                ci = i * CU + cu
                for u in range(SLICES_PER_CHUNK):
                    for j in range(NR):
                        v = cands[j][ci, pl.ds(u * LANES, LANES)]
                        accs[j] = accs[j] + jnp.maximum(v - ts[j], 0.0)
            return tuple(accs)

        accs = lax.fori_loop(0, LCAP // CU, body, (zero_v,) * NR)
        return [allsum(accs[j]) for j in range(NR)]

    for _ in range(2):
        ks = ks_at4(lo)
        lo = [(s - 1.0) / k for k, s in ks]

    f_lo = fsum4(lo)
    hi = [
        jnp.maximum(jnp.minimum(lo[j] + (f_lo[j] - 1.0), row_max[j]), lo[j])
        for j in range(NR)
    ]

    def bis(i, c):
        blo = list(c[:NR])
        bhi = list(c[NR:])
        mid = [0.5 * (blo[j] + bhi[j]) for j in range(NR)]
        fs = fsum4(mid)
        for j in range(NR):
            gt = fs[j] > 1.0
            blo[j] = jnp.where(gt, mid[j], blo[j])
            bhi[j] = jnp.where(gt, bhi[j], mid[j])
        return tuple(blo) + tuple(bhi)

    res = lax.fori_loop(0, 16, bis, tuple(lo) + tuple(hi))
    lo = list(res[:NR])

    ks = ks_at4(lo)
    tau = [(s - 1.0) / k for k, s in ks]

    d_out = [None] * NR
    for j in range(NR):
        b = j % 2
        d_in[j].wait()
        buf = rowbuf[b]
        tj = tau[j]

        def outbody(i, c):
            for cu in range(2):
                for u in range(SLICES_PER_CHUNK):
                    sl = (i * 2 + cu, pl.ds(u * LANES, LANES))
                    buf[sl] = jnp.maximum(buf[sl] - tj, 0.0)
            return c

        lax.fori_loop(0, NCHUNKS // 2, outbody, 0)
        d_out[j] = pltpu.async_copy(
            buf, out2_hbm.at[pl.ds((r0 + j) * NCHUNKS, NCHUNKS)], sem_out[b]
        )
        if j + 2 < NR:
            d_out[j].wait()
            d_in[j + 2] = pltpu.async_copy(
                x2_hbm.at[pl.ds((r0 + j + 2) * NCHUNKS, NCHUNKS)],
                rowbuf[b],
                sem_in[b],
            )
    d_out[NR - 2].wait()
    d_out[NR - 1].wait()


def _body_c(x_hbm, out_hbm, row_v):
    cid = lax.axis_index("c")
    sid = lax.axis_index("s")
    wid = sid * NUM_CORES + cid

    lane, allmax, allsum = _mk_helpers()
    ones_v, zero_v = _ONES(), _ZERO()

    def do_row(j, carry):
        r = wid * ROWS_PER_W + j
        pltpu.sync_copy(x_hbm.at[r], row_v)

        def maxbody(i, acc):
            base = i * (LANES * UNROLL)
            for u in range(UNROLL):
                acc = jnp.maximum(acc, row_v[pl.ds(base + u * LANES, LANES)])
            return acc

        acc = lax.fori_loop(0, NSLICES // UNROLL, maxbody, row_v[pl.ds(0, LANES)])
        row_max = allmax(acc)

        def ks_at(t):
            def body(i, c):
                accs = list(c)
                base = i * (LANES * UNROLL)
                for u in range(UNROLL):
                    v = row_v[pl.ds(base + u * LANES, LANES)]
                    m = v > t
                    j = u % 4
                    accs[j] = accs[j] + jnp.where(m, ones_v, zero_v)
                    accs[4 + j] = accs[4 + j] + jnp.where(m, v, zero_v)
                return tuple(accs)

            accs = lax.fori_loop(0, NSLICES // UNROLL, body, (zero_v,) * 8)
            ka = (accs[0] + accs[1]) + (accs[2] + accs[3])
            sa = (accs[4] + accs[5]) + (accs[6] + accs[7])
            return allsum(ka), allsum(sa)

        def fsum(tau):
            def body(i, c):
                accs = list(c)
                base = i * (LANES * UNROLL)
                for u in range(UNROLL):
                    v = row_v[pl.ds(base + u * LANES, LANES)]
                    accs[u % 4] = accs[u % 4] + jnp.maximum(v - tau, 0.0)
                return tuple(accs)

            accs = lax.fori_loop(0, NSLICES // UNROLL, body, (zero_v,) * 4)
            return allsum((accs[0] + accs[1]) + (accs[2] + accs[3]))

        lo = row_max - 1.001
        for _ in range(4):
            k, s = ks_at(lo)
            lo = (s - 1.0) / k

        f_lo = fsum(lo)
        hi = jnp.minimum(lo + (f_lo - 1.0), row_max)
        hi = jnp.maximum(hi, lo)

        def bis(i, c):
            blo, bhi = c
            mid = 0.5 * (blo + bhi)
            gt = fsum(mid) > 1.0
            return (jnp.where(gt, mid, blo), jnp.where(gt, bhi, mid))

        lo, _ = lax.fori_loop(0, 26, bis, (lo, hi))

        k, s = ks_at(lo)
        tau = (s - 1.0) / k

        def outbody(i, c):
            base = i * (LANES * UNROLL)
            for u in range(UNROLL):
                sl = pl.ds(base + u * LANES, LANES)
                row_v[sl] = jnp.maximum(row_v[sl] - tau, 0.0)
            return c

        lax.fori_loop(0, NSLICES // UNROLL, outbody, 0)
        pltpu.sync_copy(row_v, out_hbm.at[r])
        return carry

    lax.fori_loop(0, ROWS_PER_W, do_row, 0)


def _mesh():
    return plsc.VectorSubcoreMesh(core_axis_name="c", subcore_axis_name="s")


def _kernel_a(x):
    fn = pl.kernel(
        _body_a,
        out_type=(
            jax.ShapeDtypeStruct((ROWS, NCHUNKS * LANES), jnp.float32),
            jax.ShapeDtypeStruct((ROWS, 2 * LANES), jnp.float32),
        ),
        mesh=_mesh(),
        scratch_types=[
            pltpu.VMEM((COLS,), jnp.float32),
            pltpu.VMEM((NCHUNKS * LANES,), jnp.float32),
            pltpu.VMEM((2 * LANES,), jnp.float32),
        ],
    )
    return fn(x)


def _kernel_b(x2, idx, stats):
    fn = pl.kernel(
        _body_b,
        out_type=jax.ShapeDtypeStruct((ROWS * NCHUNKS, CHUNK), jnp.float32),
        mesh=_mesh(),
        scratch_types=[
            pltpu.VMEM((NCHUNKS, CHUNK), jnp.float32),
            pltpu.VMEM((NCHUNKS, CHUNK), jnp.float32),
            pltpu.VMEM((LCAP, CHUNK), jnp.float32),
            pltpu.VMEM((LCAP, CHUNK), jnp.float32),
            pltpu.VMEM((LCAP, CHUNK), jnp.float32),
            pltpu.VMEM((LCAP, CHUNK), jnp.float32),
            pltpu.VMEM((ROWS_PER_W, LCAP), jnp.int32),
            pltpu.VMEM((ROWS_PER_W, 2 * LANES), jnp.float32),
            pltpu.SemaphoreType.DMA,
            pltpu.SemaphoreType.DMA,
            pltpu.SemaphoreType.DMA,
            pltpu.SemaphoreType.DMA,
            pltpu.SemaphoreType.DMA,
        ],
    )
    return fn(x2, idx, stats)


def _kernel_c(x):
    fn = pl.kernel(
        _body_c,
        out_type=jax.ShapeDtypeStruct((ROWS, COLS), jnp.float32),
        mesh=_mesh(),
        scratch_types=[pltpu.VMEM((COLS,), jnp.float32)],
    )
    return fn(x)


_STAGE = 2


@jax.jit
def _pipeline(x):
    flags, stats = _kernel_a(x)
    score = flags.reshape(ROWS, NCHUNKS, LANES).max(axis=2)
    lom = stats[:, 0:1]
    flag = score > lom
    cnt = jnp.sum(flag.astype(jnp.int32), axis=1)
    overflow = jnp.any(cnt > LCAP)

    _, top_idx = lax.top_k(score, LCAP)
    base = (jnp.arange(ROWS, dtype=jnp.int32) * NCHUNKS)[:, None]
    idx = base + top_idx.astype(jnp.int32)

    x2 = x.reshape(ROWS * NCHUNKS, CHUNK)

    def fast(ops):
        xx2, iidx, sstats = ops
        out2 = _kernel_b(xx2, iidx, sstats)
        return out2.reshape(ROWS, COLS)

    def slow(ops):
        xx2, _, _ = ops
        return _kernel_c(xx2.reshape(ROWS, COLS))

    return lax.cond(overflow, slow, fast, (x2, idx, stats)) if _STAGE == 2 else ((flags, stats) if _STAGE == 0 else (flags, stats, idx, overflow))


def kernel(input):
    return _pipeline(input)


# speedup vs baseline: 1.6938x; 1.0859x over previous
import jax
import jax.numpy as jnp
from jax import lax
from jax.experimental import pallas as pl
from jax.experimental.pallas import tpu as pltpu
from jax.experimental.pallas import tpu_sc as plsc

ROWS = 128
COLS = 32768
LANES = 16
NSLICES = COLS // LANES
CHUNK = 128
NCHUNKS = COLS // CHUNK
SLICES_PER_CHUNK = CHUNK // LANES
LCAP = 48
NUM_CORES = 2
NUM_SUBCORES = 16
NWORKERS = NUM_CORES * NUM_SUBCORES
ROWS_PER_W = ROWS // NWORKERS
MARGIN = 3e-3
UNROLL = 8

_GATHER_DNUMS = lax.GatherDimensionNumbers(
    offset_dims=(), collapsed_slice_dims=(0,), start_index_map=(0,)
)


def _perm(v, idx):
    return lax.gather(
        v,
        idx[:, None],
        _GATHER_DNUMS,
        slice_sizes=(1,),
        mode=lax.GatherScatterMode.PROMISE_IN_BOUNDS,
    )


def _mk_helpers():
    lane = lax.iota(jnp.int32, LANES)
    bfly = [jnp.bitwise_xor(lane, sh) for sh in (1, 2, 4, 8)]

    def allmax(v):
        for idx in bfly:
            v = jnp.maximum(v, _perm(v, idx))
        return v

    def allsum(v):
        for idx in bfly:
            v = v + _perm(v, idx)
        return v

    return lane, allmax, allsum


_ONES = lambda: jnp.full((LANES,), 1.0, jnp.float32)
_ZERO = lambda: jnp.zeros((LANES,), jnp.float32)


def _body_a(x_hbm, flags_hbm, stats_hbm, row_v, flag_v, stats_v):
    cid = lax.axis_index("c")
    sid = lax.axis_index("s")
    wid = sid * NUM_CORES + cid

    lane, allmax, allsum = _mk_helpers()
    ones_v, zero_v = _ONES(), _ZERO()

    def do_row(j, carry):
        r = wid * ROWS_PER_W + j
        pltpu.sync_copy(x_hbm.at[r], row_v)

        def maxbody(i, acc):
            for cu in range(2):
                g = i * 2 + cu
                base = g * CHUNK
                vs = [row_v[pl.ds(base + u * LANES, LANES)] for u in range(SLICES_PER_CHUNK)]
                m0 = jnp.maximum(vs[0], vs[1])
                m1 = jnp.maximum(vs[2], vs[3])
                m2 = jnp.maximum(vs[4], vs[5])
                m3 = jnp.maximum(vs[6], vs[7])
                mx = jnp.maximum(jnp.maximum(m0, m1), jnp.maximum(m2, m3))
                flag_v[pl.ds(g * LANES, LANES)] = mx
                acc = jnp.maximum(acc, mx)
            return acc

        acc = lax.fori_loop(0, NCHUNKS // 2, maxbody, row_v[pl.ds(0, LANES)])
        row_max = allmax(acc)

        def ks_at(t):
            def body(i, c):
                accs = list(c)
                base = i * (LANES * UNROLL)
                for u in range(UNROLL):
                    v = row_v[pl.ds(base + u * LANES, LANES)]
                    m = v > t
                    j = u % 4
                    accs[j] = accs[j] + jnp.where(m, ones_v, zero_v)
                    accs[4 + j] = accs[4 + j] + jnp.where(m, v, zero_v)
                return tuple(accs)

            accs = lax.fori_loop(0, NSLICES // UNROLL, body, (zero_v,) * 8)
            ka = (accs[0] + accs[1]) + (accs[2] + accs[3])
            sa = (accs[4] + accs[5]) + (accs[6] + accs[7])
            return allsum(ka), allsum(sa)

        lo = row_max - 1.001
        for _ in range(2):
            k, s = ks_at(lo)
            lo = (s - 1.0) / k
        lom = lo - MARGIN

        stats_v[pl.ds(0, LANES)] = lom
        stats_v[pl.ds(LANES, LANES)] = row_max
        pltpu.sync_copy(flag_v, flags_hbm.at[r])
        pltpu.sync_copy(stats_v, stats_hbm.at[r])
        return carry

    lax.fori_loop(0, ROWS_PER_W, do_row, 0)


def _body_b(
    x2_hbm, idx_hbm, stats_hbm, out2_hbm,
    row2_v, rowout_v, cand_v, idx_v, stats_v,
    sem, sem_row, sem_idx, sem_st, sem_out,
):
    cid = lax.axis_index("c")
    sid = lax.axis_index("s")
    wid = sid * NUM_CORES + cid

    lane, allmax, allsum = _mk_helpers()
    ones_v, zero_v = _ONES(), _ZERO()
    CU = 4

    def do_row(j, carry):
        r = wid * ROWS_PER_W + j
        d_row = pltpu.async_copy(
            x2_hbm.at[pl.ds(r * NCHUNKS, NCHUNKS)], row2_v, sem_row
        )
        d_idx = pltpu.async_copy(idx_hbm.at[r], idx_v, sem_idx)
        d_st = pltpu.async_copy(stats_hbm.at[r], stats_v, sem_st)
        d_idx.wait()
        d_st.wait()
        pltpu.async_copy(x2_hbm.at[idx_v], cand_v, sem).wait()

        lo = stats_v[pl.ds(0, LANES)]
        row_max = stats_v[pl.ds(LANES, LANES)]

        def ks_at(t):
            def body(i, c):
                accs = list(c)
                for cu in range(CU):
                    for u in range(SLICES_PER_CHUNK):
                        v = cand_v[i * CU + cu, pl.ds(u * LANES, LANES)]
                        m = v > t
                        q = u % 4
                        accs[q] = accs[q] + jnp.where(m, ones_v, zero_v)
                        accs[4 + q] = accs[4 + q] + jnp.where(m, v, zero_v)
                return tuple(accs)

            accs = lax.fori_loop(0, LCAP // CU, body, (zero_v,) * 8)
            ka = (accs[0] + accs[1]) + (accs[2] + accs[3])
            sa = (accs[4] + accs[5]) + (accs[6] + accs[7])
            return allsum(ka), allsum(sa)

        def fsum(t):
            def body(i, c):
                accs = list(c)
                for cu in range(CU):
                    for u in range(SLICES_PER_CHUNK):
                        v = cand_v[i * CU + cu, pl.ds(u * LANES, LANES)]
                        accs[u % 4] = accs[u % 4] + jnp.maximum(v - t, 0.0)
                return tuple(accs)

            accs = lax.fori_loop(0, LCAP // CU, body, (zero_v,) * 4)
            return allsum((accs[0] + accs[1]) + (accs[2] + accs[3]))

        for _ in range(3):
            k, s = ks_at(lo)
            lo = (s - 1.0) / k

        f_lo = fsum(lo)
        hi = jnp.minimum(lo + (f_lo - 1.0), row_max)
        hi = jnp.maximum(hi, lo)

        def bis(i, c):
            blo, bhi = c
            mid = 0.5 * (blo + bhi)
            gt = fsum(mid) > 1.0
            return (jnp.where(gt, mid, blo), jnp.where(gt, bhi, mid))

        lo, _ = lax.fori_loop(0, 12, bis, (lo, hi))

        k, s = ks_at(lo)
        tau = (s - 1.0) / k

        d_row.wait()

        def outbody(i, c):
            for cu in range(2):
                for u in range(SLICES_PER_CHUNK):
                    src = (i * 2 + cu, pl.ds(u * LANES, LANES))
                    rowout_v[src] = jnp.maximum(row2_v[src] - tau, 0.0)
            return c

        lax.fori_loop(0, NCHUNKS // 2, outbody, 0)
        d_out = pltpu.async_copy(
            rowout_v, out2_hbm.at[pl.ds(r * NCHUNKS, NCHUNKS)], sem_out
        )
        d_out.wait()
        return carry

    lax.fori_loop(0, ROWS_PER_W, do_row, 0)


def _body_c(x_hbm, out_hbm, row_v):
    cid = lax.axis_index("c")
    sid = lax.axis_index("s")
    wid = sid * NUM_CORES + cid

    lane, allmax, allsum = _mk_helpers()
    ones_v, zero_v = _ONES(), _ZERO()

    def do_row(j, carry):
        r = wid * ROWS_PER_W + j
        pltpu.sync_copy(x_hbm.at[r], row_v)

        def maxbody(i, acc):
            base = i * (LANES * UNROLL)
            for u in range(UNROLL):
                acc = jnp.maximum(acc, row_v[pl.ds(base + u * LANES, LANES)])
            return acc

        acc = lax.fori_loop(0, NSLICES // UNROLL, maxbody, row_v[pl.ds(0, LANES)])
        row_max = allmax(acc)

        def ks_at(t):
            def body(i, c):
                accs = list(c)
                base = i * (LANES * UNROLL)
                for u in range(UNROLL):
                    v = row_v[pl.ds(base + u * LANES, LANES)]
                    m = v > t
                    j = u % 4
                    accs[j] = accs[j] + jnp.where(m, ones_v, zero_v)
                    accs[4 + j] = accs[4 + j] + jnp.where(m, v, zero_v)
                return tuple(accs)

            accs = lax.fori_loop(0, NSLICES // UNROLL, body, (zero_v,) * 8)
            ka = (accs[0] + accs[1]) + (accs[2] + accs[3])
            sa = (accs[4] + accs[5]) + (accs[6] + accs[7])
            return allsum(ka), allsum(sa)

        def fsum(tau):
            def body(i, c):
                accs = list(c)
                base = i * (LANES * UNROLL)
                for u in range(UNROLL):
                    v = row_v[pl.ds(base + u * LANES, LANES)]
                    accs[u % 4] = accs[u % 4] + jnp.maximum(v - tau, 0.0)
                return tuple(accs)

            accs = lax.fori_loop(0, NSLICES // UNROLL, body, (zero_v,) * 4)
            return allsum((accs[0] + accs[1]) + (accs[2] + accs[3]))

        lo = row_max - 1.001
        for _ in range(4):
            k, s = ks_at(lo)
            lo = (s - 1.0) / k

        f_lo = fsum(lo)
        hi = jnp.minimum(lo + (f_lo - 1.0), row_max)
        hi = jnp.maximum(hi, lo)

        def bis(i, c):
            blo, bhi = c
            mid = 0.5 * (blo + bhi)
            gt = fsum(mid) > 1.0
            return (jnp.where(gt, mid, blo), jnp.where(gt, bhi, mid))

        lo, _ = lax.fori_loop(0, 26, bis, (lo, hi))

        k, s = ks_at(lo)
        tau = (s - 1.0) / k

        def outbody(i, c):
            base = i * (LANES * UNROLL)
            for u in range(UNROLL):
                sl = pl.ds(base + u * LANES, LANES)
                row_v[sl] = jnp.maximum(row_v[sl] - tau, 0.0)
            return c

        lax.fori_loop(0, NSLICES // UNROLL, outbody, 0)
        pltpu.sync_copy(row_v, out_hbm.at[r])
        return carry

    lax.fori_loop(0, ROWS_PER_W, do_row, 0)


def _mesh():
    return plsc.VectorSubcoreMesh(core_axis_name="c", subcore_axis_name="s")


def _kernel_a(x):
    fn = pl.kernel(
        _body_a,
        out_type=(
            jax.ShapeDtypeStruct((ROWS, NCHUNKS * LANES), jnp.float32),
            jax.ShapeDtypeStruct((ROWS, 2 * LANES), jnp.float32),
        ),
        mesh=_mesh(),
        scratch_types=[
            pltpu.VMEM((COLS,), jnp.float32),
            pltpu.VMEM((NCHUNKS * LANES,), jnp.float32),
            pltpu.VMEM((2 * LANES,), jnp.float32),
        ],
    )
    return fn(x)


def _kernel_b(x2, idx, stats):
    fn = pl.kernel(
        _body_b,
        out_type=jax.ShapeDtypeStruct((ROWS * NCHUNKS, CHUNK), jnp.float32),
        mesh=_mesh(),
        scratch_types=[
            pltpu.VMEM((NCHUNKS, CHUNK), jnp.float32),
            pltpu.VMEM((NCHUNKS, CHUNK), jnp.float32),
            pltpu.VMEM((LCAP, CHUNK), jnp.float32),
            pltpu.VMEM((LCAP,), jnp.int32),
            pltpu.VMEM((2 * LANES,), jnp.float32),
            pltpu.SemaphoreType.DMA,
            pltpu.SemaphoreType.DMA,
            pltpu.SemaphoreType.DMA,
            pltpu.SemaphoreType.DMA,
            pltpu.SemaphoreType.DMA,
        ],
    )
    return fn(x2, idx, stats)


def _kernel_c(x):
    fn = pl.kernel(
        _body_c,
        out_type=jax.ShapeDtypeStruct((ROWS, COLS), jnp.float32),
        mesh=_mesh(),
        scratch_types=[pltpu.VMEM((COLS,), jnp.float32)],
    )
    return fn(x)


_STAGE = 2


@jax.jit
def _pipeline(x):
    flags, stats = _kernel_a(x)
    score = flags.reshape(ROWS, NCHUNKS, LANES).max(axis=2)
    lom = stats[:, 0:1]
    flag = score > lom
    cnt = jnp.sum(flag.astype(jnp.int32), axis=1)
    overflow = jnp.any(cnt > LCAP)

    _, top_idx = lax.top_k(score, LCAP)
    base = (jnp.arange(ROWS, dtype=jnp.int32) * NCHUNKS)[:, None]
    idx = base + top_idx.astype(jnp.int32)

    x2 = x.reshape(ROWS * NCHUNKS, CHUNK)

    def fast(ops):
        xx2, iidx, sstats = ops
        out2 = _kernel_b(xx2, iidx, sstats)
        return out2.reshape(ROWS, COLS)

    def slow(ops):
        xx2, _, _ = ops
        return _kernel_c(xx2.reshape(ROWS, COLS))

    return lax.cond(overflow, slow, fast, (x2, idx, stats)) if _STAGE == 2 else ((flags, stats) if _STAGE == 0 else (flags, stats, idx, overflow))


def kernel(input):
    return _pipeline(input)


# speedup vs baseline: 1.7592x; 1.0386x over previous
import jax
import jax.numpy as jnp
from jax import lax
from jax.experimental import pallas as pl
from jax.experimental.pallas import tpu as pltpu
from jax.experimental.pallas import tpu_sc as plsc

ROWS = 128
COLS = 32768
LANES = 16
NSLICES = COLS // LANES
CHUNK = 128
NCHUNKS = COLS // CHUNK
SLICES_PER_CHUNK = CHUNK // LANES
LCAP = 48
NUM_CORES = 2
NUM_SUBCORES = 16
NWORKERS = NUM_CORES * NUM_SUBCORES
ROWS_PER_W = ROWS // NWORKERS
MARGIN = 3e-3
UNROLL = 8

_GATHER_DNUMS = lax.GatherDimensionNumbers(
    offset_dims=(), collapsed_slice_dims=(0,), start_index_map=(0,)
)


def _perm(v, idx):
    return lax.gather(
        v,
        idx[:, None],
        _GATHER_DNUMS,
        slice_sizes=(1,),
        mode=lax.GatherScatterMode.PROMISE_IN_BOUNDS,
    )


def _mk_helpers():
    lane = lax.iota(jnp.int32, LANES)
    bfly = [jnp.bitwise_xor(lane, sh) for sh in (1, 2, 4, 8)]

    def allmax(v):
        for idx in bfly:
            v = jnp.maximum(v, _perm(v, idx))
        return v

    def allsum(v):
        for idx in bfly:
            v = v + _perm(v, idx)
        return v

    return lane, allmax, allsum


_ONES = lambda: jnp.full((LANES,), 1.0, jnp.float32)
_ZERO = lambda: jnp.zeros((LANES,), jnp.float32)


def _body_a(x_hbm, flags_hbm, stats_hbm, rowa_v, rowb_v, flag_v, stats_v, sem_a, sem_b):
    cid = lax.axis_index("c")
    sid = lax.axis_index("s")
    wid = sid * NUM_CORES + cid
    r0 = wid * ROWS_PER_W

    lane, allmax, allsum = _mk_helpers()
    ones_v, zero_v = _ONES(), _ZERO()
    rowbuf = [rowa_v, rowb_v]
    sems = [sem_a, sem_b]

    d_in = [None] * ROWS_PER_W
    d_in[0] = pltpu.async_copy(x_hbm.at[r0], rowa_v, sem_a)

    for j in range(ROWS_PER_W):
        row_v = rowbuf[j % 2]
        d_in[j].wait()
        if j + 1 < ROWS_PER_W:
            d_in[j + 1] = pltpu.async_copy(
                x_hbm.at[r0 + j + 1], rowbuf[(j + 1) % 2], sems[(j + 1) % 2]
            )

        def maxbody(i, acc):
            for cu in range(2):
                g = i * 2 + cu
                base = g * CHUNK
                vs = [row_v[pl.ds(base + u * LANES, LANES)] for u in range(SLICES_PER_CHUNK)]
                m0 = jnp.maximum(vs[0], vs[1])
                m1 = jnp.maximum(vs[2], vs[3])
                m2 = jnp.maximum(vs[4], vs[5])
                m3 = jnp.maximum(vs[6], vs[7])
                mx = jnp.maximum(jnp.maximum(m0, m1), jnp.maximum(m2, m3))
                flag_v[pl.ds(g * LANES, LANES)] = mx
                acc = jnp.maximum(acc, mx)
            return acc

        acc = lax.fori_loop(0, NCHUNKS // 2, maxbody, row_v[pl.ds(0, LANES)])
        row_max = allmax(acc)

        def ks_at(t):
            def body(i, c):
                accs = list(c)
                base = i * (LANES * UNROLL)
                for u in range(UNROLL):
                    v = row_v[pl.ds(base + u * LANES, LANES)]
                    m = v > t
                    q = u % 4
                    accs[q] = accs[q] + jnp.where(m, ones_v, zero_v)
                    accs[4 + q] = accs[4 + q] + jnp.where(m, v, zero_v)
                return tuple(accs)

            accs = lax.fori_loop(0, NSLICES // UNROLL, body, (zero_v,) * 8)
            ka = (accs[0] + accs[1]) + (accs[2] + accs[3])
            sa = (accs[4] + accs[5]) + (accs[6] + accs[7])
            return allsum(ka), allsum(sa)

        lo = row_max - 1.001
        for _ in range(2):
            k, s = ks_at(lo)
            lo = (s - 1.0) / k
        lom = lo - MARGIN

        stats_v[pl.ds(0, LANES)] = lom
        stats_v[pl.ds(LANES, LANES)] = row_max
        pltpu.sync_copy(flag_v, flags_hbm.at[r0 + j])
        pltpu.sync_copy(stats_v, stats_hbm.at[r0 + j])


def _body_b(
    x2_hbm, idx_hbm, stats_hbm, out2_hbm,
    row2_v, rowout_v, cand_v, idx_v, stats_v,
    sem, sem_row, sem_idx, sem_st, sem_out,
):
    cid = lax.axis_index("c")
    sid = lax.axis_index("s")
    wid = sid * NUM_CORES + cid

    lane, allmax, allsum = _mk_helpers()
    ones_v, zero_v = _ONES(), _ZERO()
    CU = 4

    def do_row(j, carry):
        r = wid * ROWS_PER_W + j
        d_row = pltpu.async_copy(
            x2_hbm.at[pl.ds(r * NCHUNKS, NCHUNKS)], row2_v, sem_row
        )
        d_idx = pltpu.async_copy(idx_hbm.at[r], idx_v, sem_idx)
        d_st = pltpu.async_copy(stats_hbm.at[r], stats_v, sem_st)
        d_idx.wait()
        d_st.wait()
        pltpu.async_copy(x2_hbm.at[idx_v], cand_v, sem).wait()

        lo = stats_v[pl.ds(0, LANES)]
        row_max = stats_v[pl.ds(LANES, LANES)]

        def ks_at(t):
            def body(i, c):
                accs = list(c)
                for cu in range(CU):
                    for u in range(SLICES_PER_CHUNK):
                        v = cand_v[i * CU + cu, pl.ds(u * LANES, LANES)]
                        m = v > t
                        q = u % 4
                        accs[q] = accs[q] + jnp.where(m, ones_v, zero_v)
                        accs[4 + q] = accs[4 + q] + jnp.where(m, v, zero_v)
                return tuple(accs)

            accs = lax.fori_loop(0, LCAP // CU, body, (zero_v,) * 8)
            ka = (accs[0] + accs[1]) + (accs[2] + accs[3])
            sa = (accs[4] + accs[5]) + (accs[6] + accs[7])
            return allsum(ka), allsum(sa)

        def fsum(t):
            def body(i, c):
                accs = list(c)
                for cu in range(CU):
                    for u in range(SLICES_PER_CHUNK):
                        v = cand_v[i * CU + cu, pl.ds(u * LANES, LANES)]
                        accs[u % 4] = accs[u % 4] + jnp.maximum(v - t, 0.0)
                return tuple(accs)

            accs = lax.fori_loop(0, LCAP // CU, body, (zero_v,) * 4)
            return allsum((accs[0] + accs[1]) + (accs[2] + accs[3]))

        for _ in range(3):
            k, s = ks_at(lo)
            lo = (s - 1.0) / k

        f_lo = fsum(lo)
        hi = jnp.minimum(lo + (f_lo - 1.0), row_max)
        hi = jnp.maximum(hi, lo)

        def bis(i, c):
            blo, bhi = c
            mid = 0.5 * (blo + bhi)
            gt = fsum(mid) > 1.0
            return (jnp.where(gt, mid, blo), jnp.where(gt, bhi, mid))

        lo, _ = lax.fori_loop(0, 12, bis, (lo, hi))

        k, s = ks_at(lo)
        tau = (s - 1.0) / k

        d_row.wait()

        def outbody(i, c):
            for cu in range(2):
                for u in range(SLICES_PER_CHUNK):
                    src = (i * 2 + cu, pl.ds(u * LANES, LANES))
                    rowout_v[src] = jnp.maximum(row2_v[src] - tau, 0.0)
            return c

        lax.fori_loop(0, NCHUNKS // 2, outbody, 0)
        d_out = pltpu.async_copy(
            rowout_v, out2_hbm.at[pl.ds(r * NCHUNKS, NCHUNKS)], sem_out
        )
        d_out.wait()
        return carry

    lax.fori_loop(0, ROWS_PER_W, do_row, 0)


def _body_c(x_hbm, out_hbm, row_v):
    cid = lax.axis_index("c")
    sid = lax.axis_index("s")
    wid = sid * NUM_CORES + cid

    lane, allmax, allsum = _mk_helpers()
    ones_v, zero_v = _ONES(), _ZERO()

    def do_row(j, carry):
        r = wid * ROWS_PER_W + j
        pltpu.sync_copy(x_hbm.at[r], row_v)

        def maxbody(i, acc):
            base = i * (LANES * UNROLL)
            for u in range(UNROLL):
                acc = jnp.maximum(acc, row_v[pl.ds(base + u * LANES, LANES)])
            return acc

        acc = lax.fori_loop(0, NSLICES // UNROLL, maxbody, row_v[pl.ds(0, LANES)])
        row_max = allmax(acc)

        def ks_at(t):
            def body(i, c):
                accs = list(c)
                base = i * (LANES * UNROLL)
                for u in range(UNROLL):
                    v = row_v[pl.ds(base + u * LANES, LANES)]
                    m = v > t
                    j = u % 4
                    accs[j] = accs[j] + jnp.where(m, ones_v, zero_v)
                    accs[4 + j] = accs[4 + j] + jnp.where(m, v, zero_v)
                return tuple(accs)

            accs = lax.fori_loop(0, NSLICES // UNROLL, body, (zero_v,) * 8)
            ka = (accs[0] + accs[1]) + (accs[2] + accs[3])
            sa = (accs[4] + accs[5]) + (accs[6] + accs[7])
            return allsum(ka), allsum(sa)

        def fsum(tau):
            def body(i, c):
                accs = list(c)
                base = i * (LANES * UNROLL)
                for u in range(UNROLL):
                    v = row_v[pl.ds(base + u * LANES, LANES)]
                    accs[u % 4] = accs[u % 4] + jnp.maximum(v - tau, 0.0)
                return tuple(accs)

            accs = lax.fori_loop(0, NSLICES // UNROLL, body, (zero_v,) * 4)
            return allsum((accs[0] + accs[1]) + (accs[2] + accs[3]))

        lo = row_max - 1.001
        for _ in range(4):
            k, s = ks_at(lo)
            lo = (s - 1.0) / k

        f_lo = fsum(lo)
        hi = jnp.minimum(lo + (f_lo - 1.0), row_max)
        hi = jnp.maximum(hi, lo)

        def bis(i, c):
            blo, bhi = c
            mid = 0.5 * (blo + bhi)
            gt = fsum(mid) > 1.0
            return (jnp.where(gt, mid, blo), jnp.where(gt, bhi, mid))

        lo, _ = lax.fori_loop(0, 26, bis, (lo, hi))

        k, s = ks_at(lo)
        tau = (s - 1.0) / k

        def outbody(i, c):
            base = i * (LANES * UNROLL)
            for u in range(UNROLL):
                sl = pl.ds(base + u * LANES, LANES)
                row_v[sl] = jnp.maximum(row_v[sl] - tau, 0.0)
            return c

        lax.fori_loop(0, NSLICES // UNROLL, outbody, 0)
        pltpu.sync_copy(row_v, out_hbm.at[r])
        return carry

    lax.fori_loop(0, ROWS_PER_W, do_row, 0)


def _mesh():
    return plsc.VectorSubcoreMesh(core_axis_name="c", subcore_axis_name="s")


def _kernel_a(x):
    fn = pl.kernel(
        _body_a,
        out_type=(
            jax.ShapeDtypeStruct((ROWS, NCHUNKS * LANES), jnp.float32),
            jax.ShapeDtypeStruct((ROWS, 2 * LANES), jnp.float32),
        ),
        mesh=_mesh(),
        scratch_types=[
            pltpu.VMEM((COLS,), jnp.float32),
            pltpu.VMEM((COLS,), jnp.float32),
            pltpu.VMEM((NCHUNKS * LANES,), jnp.float32),
            pltpu.VMEM((2 * LANES,), jnp.float32),
            pltpu.SemaphoreType.DMA,
            pltpu.SemaphoreType.DMA,
        ],
    )
    return fn(x)


def _kernel_b(x2, idx, stats):
    fn = pl.kernel(
        _body_b,
        out_type=jax.ShapeDtypeStruct((ROWS * NCHUNKS, CHUNK), jnp.float32),
        mesh=_mesh(),
        scratch_types=[
            pltpu.VMEM((NCHUNKS, CHUNK), jnp.float32),
            pltpu.VMEM((NCHUNKS, CHUNK), jnp.float32),
            pltpu.VMEM((LCAP, CHUNK), jnp.float32),
            pltpu.VMEM((LCAP,), jnp.int32),
            pltpu.VMEM((2 * LANES,), jnp.float32),
            pltpu.SemaphoreType.DMA,
            pltpu.SemaphoreType.DMA,
            pltpu.SemaphoreType.DMA,
            pltpu.SemaphoreType.DMA,
            pltpu.SemaphoreType.DMA,
        ],
    )
    return fn(x2, idx, stats)


def _kernel_c(x):
    fn = pl.kernel(
        _body_c,
        out_type=jax.ShapeDtypeStruct((ROWS, COLS), jnp.float32),
        mesh=_mesh(),
        scratch_types=[pltpu.VMEM((COLS,), jnp.float32)],
    )
    return fn(x)


_STAGE = 2


@jax.jit
def _pipeline(x):
    flags, stats = _kernel_a(x)
    score = flags.reshape(ROWS, NCHUNKS, LANES).max(axis=2)
    lom = stats[:, 0:1]
    flag = score > lom
    cnt = jnp.sum(flag.astype(jnp.int32), axis=1)
    overflow = jnp.any(cnt > LCAP)

    _, top_idx = lax.top_k(score, LCAP)
    base = (jnp.arange(ROWS, dtype=jnp.int32) * NCHUNKS)[:, None]
    idx = base + top_idx.astype(jnp.int32)

    x2 = x.reshape(ROWS * NCHUNKS, CHUNK)

    def fast(ops):
        xx2, iidx, sstats = ops
        out2 = _kernel_b(xx2, iidx, sstats)
        return out2.reshape(ROWS, COLS)

    def slow(ops):
        xx2, _, _ = ops
        return _kernel_c(xx2.reshape(ROWS, COLS))

    return lax.cond(overflow, slow, fast, (x2, idx, stats)) if _STAGE == 2 else ((flags, stats) if _STAGE == 0 else (flags, stats, idx, overflow))


def kernel(input):
    return _pipeline(input)


# speedup vs baseline: 1.8337x; 1.0424x over previous
import jax
import jax.numpy as jnp
from jax import lax
from jax.experimental import pallas as pl
from jax.experimental.pallas import tpu as pltpu
from jax.experimental.pallas import tpu_sc as plsc

ROWS = 128
COLS = 32768
LANES = 16
NSLICES = COLS // LANES
CHUNK = 128
NCHUNKS = COLS // CHUNK
SLICES_PER_CHUNK = CHUNK // LANES
LCAP = 48
NUM_CORES = 2
NUM_SUBCORES = 16
NWORKERS = NUM_CORES * NUM_SUBCORES
ROWS_PER_W = ROWS // NWORKERS
MARGIN = 3e-3
UNROLL = 8

_GATHER_DNUMS = lax.GatherDimensionNumbers(
    offset_dims=(), collapsed_slice_dims=(0,), start_index_map=(0,)
)


def _perm(v, idx):
    return lax.gather(
        v,
        idx[:, None],
        _GATHER_DNUMS,
        slice_sizes=(1,),
        mode=lax.GatherScatterMode.PROMISE_IN_BOUNDS,
    )


def _mk_helpers():
    lane = lax.iota(jnp.int32, LANES)
    bfly = [jnp.bitwise_xor(lane, sh) for sh in (1, 2, 4, 8)]

    def allmax(v):
        for idx in bfly:
            v = jnp.maximum(v, _perm(v, idx))
        return v

    def allsum(v):
        for idx in bfly:
            v = v + _perm(v, idx)
        return v

    return lane, allmax, allsum


_ONES = lambda: jnp.full((LANES,), 1.0, jnp.float32)
_ZERO = lambda: jnp.zeros((LANES,), jnp.float32)


def _body_a(x_hbm, flags_hbm, stats_hbm, rowa_v, rowb_v, flag_v, stats_v, sem_a, sem_b):
    cid = lax.axis_index("c")
    sid = lax.axis_index("s")
    wid = sid * NUM_CORES + cid
    r0 = wid * ROWS_PER_W

    lane, allmax, allsum = _mk_helpers()
    ones_v, zero_v = _ONES(), _ZERO()
    rowbuf = [rowa_v, rowb_v]
    sems = [sem_a, sem_b]

    d_in = [None] * ROWS_PER_W
    d_in[0] = pltpu.async_copy(x_hbm.at[r0], rowa_v, sem_a)

    for j in range(ROWS_PER_W):
        row_v = rowbuf[j % 2]
        d_in[j].wait()
        if j + 1 < ROWS_PER_W:
            d_in[j + 1] = pltpu.async_copy(
                x_hbm.at[r0 + j + 1], rowbuf[(j + 1) % 2], sems[(j + 1) % 2]
            )

        def maxbody(i, acc):
            for cu in range(2):
                g = i * 2 + cu
                base = g * CHUNK
                vs = [row_v[pl.ds(base + u * LANES, LANES)] for u in range(SLICES_PER_CHUNK)]
                m0 = jnp.maximum(vs[0], vs[1])
                m1 = jnp.maximum(vs[2], vs[3])
                m2 = jnp.maximum(vs[4], vs[5])
                m3 = jnp.maximum(vs[6], vs[7])
                mx = jnp.maximum(jnp.maximum(m0, m1), jnp.maximum(m2, m3))
                flag_v[pl.ds(g * LANES, LANES)] = mx
                acc = jnp.maximum(acc, mx)
            return acc

        acc = lax.fori_loop(0, NCHUNKS // 2, maxbody, row_v[pl.ds(0, LANES)])
        row_max = allmax(acc)

        def ks_at(t):
            def body(i, c):
                accs = list(c)
                base = i * (LANES * UNROLL)
                for u in range(UNROLL):
                    v = row_v[pl.ds(base + u * LANES, LANES)]
                    m = v > t
                    q = u % 4
                    accs[q] = accs[q] + jnp.where(m, ones_v, zero_v)
                    accs[4 + q] = accs[4 + q] + jnp.where(m, v, zero_v)
                return tuple(accs)

            accs = lax.fori_loop(0, NSLICES // UNROLL, body, (zero_v,) * 8)
            ka = (accs[0] + accs[1]) + (accs[2] + accs[3])
            sa = (accs[4] + accs[5]) + (accs[6] + accs[7])
            return allsum(ka), allsum(sa)

        lo = row_max - 1.001
        for _ in range(2):
            k, s = ks_at(lo)
            lo = (s - 1.0) / k
        lom = lo - MARGIN

        stats_v[pl.ds(0, LANES)] = lom
        stats_v[pl.ds(LANES, LANES)] = row_max
        pltpu.sync_copy(flag_v, flags_hbm.at[r0 + j])
        pltpu.sync_copy(stats_v, stats_hbm.at[r0 + j])


def _body_b(
    x2_hbm, idx_hbm, stats_hbm, out2_hbm,
    rowin0, rowin1, cand_a, cand_b, cand_c, cand_d, idx4_v, stats4_v,
    sem_g, sem_i0, sem_i1, sem_o0, sem_o1,
):
    cid = lax.axis_index("c")
    sid = lax.axis_index("s")
    wid = sid * NUM_CORES + cid
    r0 = wid * ROWS_PER_W

    lane, allmax, allsum = _mk_helpers()
    ones_v, zero_v = _ONES(), _ZERO()
    NR = ROWS_PER_W
    CU = 4
    cands = [cand_a, cand_b, cand_c, cand_d]
    rowin = [rowin0, rowin1]
    sem_in = [sem_i0, sem_i1]
    sem_out = [sem_o0, sem_o1]

    d_in = [None] * NR
    d_out = [None] * NR
    for j in range(2):
        d_in[j] = pltpu.async_copy(
            x2_hbm.at[pl.ds((r0 + j) * NCHUNKS, NCHUNKS)], rowin[j], sem_in[j]
        )
    pltpu.sync_copy(idx_hbm.at[pl.ds(r0, NR)], idx4_v)
    pltpu.sync_copy(stats_hbm.at[pl.ds(r0, NR)], stats4_v)
    d_g = [
        pltpu.async_copy(x2_hbm.at[idx4_v.at[j]], cands[j], sem_g)
        for j in range(NR)
    ]

    for j in range(NR):
        cand_v = cands[j]
        d_g[j].wait()
        lo = stats4_v[j, pl.ds(0, LANES)]
        row_max = stats4_v[j, pl.ds(LANES, LANES)]

        def ks_at(t):
            def body(i, c):
                accs = list(c)
                for cu in range(CU):
                    for u in range(SLICES_PER_CHUNK):
                        v = cand_v[i * CU + cu, pl.ds(u * LANES, LANES)]
                        m = v > t
                        q = u % 4
                        accs[q] = accs[q] + jnp.where(m, ones_v, zero_v)
                        accs[4 + q] = accs[4 + q] + jnp.where(m, v, zero_v)
                return tuple(accs)

            accs = lax.fori_loop(0, LCAP // CU, body, (zero_v,) * 8)
            ka = (accs[0] + accs[1]) + (accs[2] + accs[3])
            sa = (accs[4] + accs[5]) + (accs[6] + accs[7])
            return allsum(ka), allsum(sa)

        def fsum(t):
            def body(i, c):
                accs = list(c)
                for cu in range(CU):
                    for u in range(SLICES_PER_CHUNK):
                        v = cand_v[i * CU + cu, pl.ds(u * LANES, LANES)]
                        accs[u % 4] = accs[u % 4] + jnp.maximum(v - t, 0.0)
                return tuple(accs)

            accs = lax.fori_loop(0, LCAP // CU, body, (zero_v,) * 4)
            return allsum((accs[0] + accs[1]) + (accs[2] + accs[3]))

        for _ in range(3):
            k, s = ks_at(lo)
            lo = (s - 1.0) / k

        f_lo = fsum(lo)
        hi = jnp.minimum(lo + (f_lo - 1.0), row_max)
        hi = jnp.maximum(hi, lo)

        def bis(i, c):
            blo, bhi = c
            mid = 0.5 * (blo + bhi)
            gt = fsum(mid) > 1.0
            return (jnp.where(gt, mid, blo), jnp.where(gt, bhi, mid))

        lo, _ = lax.fori_loop(0, 12, bis, (lo, hi))

        k, s = ks_at(lo)
        tau = (s - 1.0) / k

        if j >= 1 and j + 1 < NR:
            d_out[j - 1].wait()
            d_in[j + 1] = pltpu.async_copy(
                x2_hbm.at[pl.ds((r0 + j + 1) * NCHUNKS, NCHUNKS)],
                rowin[(j + 1) % 2],
                sem_in[(j + 1) % 2],
            )

        d_in[j].wait()
        buf = rowin[j % 2]

        def outbody(i, c):
            for cu in range(2):
                for u in range(SLICES_PER_CHUNK):
                    sl = (i * 2 + cu, pl.ds(u * LANES, LANES))
                    buf[sl] = jnp.maximum(buf[sl] - tau, 0.0)
            return c

        lax.fori_loop(0, NCHUNKS // 2, outbody, 0)
        d_out[j] = pltpu.async_copy(
            buf, out2_hbm.at[pl.ds((r0 + j) * NCHUNKS, NCHUNKS)], sem_out[j % 2]
        )

    d_out[NR - 2].wait()
    d_out[NR - 1].wait()


def _body_c(x_hbm, out_hbm, row_v):
    cid = lax.axis_index("c")
    sid = lax.axis_index("s")
    wid = sid * NUM_CORES + cid

    lane, allmax, allsum = _mk_helpers()
    ones_v, zero_v = _ONES(), _ZERO()

    def do_row(j, carry):
        r = wid * ROWS_PER_W + j
        pltpu.sync_copy(x_hbm.at[r], row_v)

        def maxbody(i, acc):
            base = i * (LANES * UNROLL)
            for u in range(UNROLL):
                acc = jnp.maximum(acc, row_v[pl.ds(base + u * LANES, LANES)])
            return acc

        acc = lax.fori_loop(0, NSLICES // UNROLL, maxbody, row_v[pl.ds(0, LANES)])
        row_max = allmax(acc)

        def ks_at(t):
            def body(i, c):
                accs = list(c)
                base = i * (LANES * UNROLL)
                for u in range(UNROLL):
                    v = row_v[pl.ds(base + u * LANES, LANES)]
                    m = v > t
                    j = u % 4
                    accs[j] = accs[j] + jnp.where(m, ones_v, zero_v)
                    accs[4 + j] = accs[4 + j] + jnp.where(m, v, zero_v)
                return tuple(accs)

            accs = lax.fori_loop(0, NSLICES // UNROLL, body, (zero_v,) * 8)
            ka = (accs[0] + accs[1]) + (accs[2] + accs[3])
            sa = (accs[4] + accs[5]) + (accs[6] + accs[7])
            return allsum(ka), allsum(sa)

        def fsum(tau):
            def body(i, c):
                accs = list(c)
                base = i * (LANES * UNROLL)
                for u in range(UNROLL):
                    v = row_v[pl.ds(base + u * LANES, LANES)]
                    accs[u % 4] = accs[u % 4] + jnp.maximum(v - tau, 0.0)
                return tuple(accs)

            accs = lax.fori_loop(0, NSLICES // UNROLL, body, (zero_v,) * 4)
            return allsum((accs[0] + accs[1]) + (accs[2] + accs[3]))

        lo = row_max - 1.001
        for _ in range(4):
            k, s = ks_at(lo)
            lo = (s - 1.0) / k

        f_lo = fsum(lo)
        hi = jnp.minimum(lo + (f_lo - 1.0), row_max)
        hi = jnp.maximum(hi, lo)

        def bis(i, c):
            blo, bhi = c
            mid = 0.5 * (blo + bhi)
            gt = fsum(mid) > 1.0
            return (jnp.where(gt, mid, blo), jnp.where(gt, bhi, mid))

        lo, _ = lax.fori_loop(0, 26, bis, (lo, hi))

        k, s = ks_at(lo)
        tau = (s - 1.0) / k

        def outbody(i, c):
            base = i * (LANES * UNROLL)
            for u in range(UNROLL):
                sl = pl.ds(base + u * LANES, LANES)
                row_v[sl] = jnp.maximum(row_v[sl] - tau, 0.0)
            return c

        lax.fori_loop(0, NSLICES // UNROLL, outbody, 0)
        pltpu.sync_copy(row_v, out_hbm.at[r])
        return carry

    lax.fori_loop(0, ROWS_PER_W, do_row, 0)


def _mesh():
    return plsc.VectorSubcoreMesh(core_axis_name="c", subcore_axis_name="s")


def _kernel_a(x):
    fn = pl.kernel(
        _body_a,
        out_type=(
            jax.ShapeDtypeStruct((ROWS, NCHUNKS * LANES), jnp.float32),
            jax.ShapeDtypeStruct((ROWS, 2 * LANES), jnp.float32),
        ),
        mesh=_mesh(),
        scratch_types=[
            pltpu.VMEM((COLS,), jnp.float32),
            pltpu.VMEM((COLS,), jnp.float32),
            pltpu.VMEM((NCHUNKS * LANES,), jnp.float32),
            pltpu.VMEM((2 * LANES,), jnp.float32),
            pltpu.SemaphoreType.DMA,
            pltpu.SemaphoreType.DMA,
        ],
    )
    return fn(x)


def _kernel_b(x2, idx, stats):
    fn = pl.kernel(
        _body_b,
        out_type=jax.ShapeDtypeStruct((ROWS * NCHUNKS, CHUNK), jnp.float32),
        mesh=_mesh(),
        scratch_types=[
            pltpu.VMEM((NCHUNKS, CHUNK), jnp.float32),
            pltpu.VMEM((NCHUNKS, CHUNK), jnp.float32),
            pltpu.VMEM((LCAP, CHUNK), jnp.float32),
            pltpu.VMEM((LCAP, CHUNK), jnp.float32),
            pltpu.VMEM((LCAP, CHUNK), jnp.float32),
            pltpu.VMEM((LCAP, CHUNK), jnp.float32),
            pltpu.VMEM((ROWS_PER_W, LCAP), jnp.int32),
            pltpu.VMEM((ROWS_PER_W, 2 * LANES), jnp.float32),
            pltpu.SemaphoreType.DMA,
            pltpu.SemaphoreType.DMA,
            pltpu.SemaphoreType.DMA,
            pltpu.SemaphoreType.DMA,
            pltpu.SemaphoreType.DMA,
        ],
    )
    return fn(x2, idx, stats)


def _kernel_c(x):
    fn = pl.kernel(
        _body_c,
        out_type=jax.ShapeDtypeStruct((ROWS, COLS), jnp.float32),
        mesh=_mesh(),
        scratch_types=[pltpu.VMEM((COLS,), jnp.float32)],
    )
    return fn(x)


_STAGE = 2


@jax.jit
def _pipeline(x):
    flags, stats = _kernel_a(x)
    score = flags.reshape(ROWS, NCHUNKS, LANES).max(axis=2)
    lom = stats[:, 0:1]
    flag = score > lom
    cnt = jnp.sum(flag.astype(jnp.int32), axis=1)
    overflow = jnp.any(cnt > LCAP)

    _, top_idx = lax.top_k(score, LCAP)
    base = (jnp.arange(ROWS, dtype=jnp.int32) * NCHUNKS)[:, None]
    idx = base + top_idx.astype(jnp.int32)

    x2 = x.reshape(ROWS * NCHUNKS, CHUNK)

    def fast(ops):
        xx2, iidx, sstats = ops
        out2 = _kernel_b(xx2, iidx, sstats)
        return out2.reshape(ROWS, COLS)

    def slow(ops):
        xx2, _, _ = ops
        return _kernel_c(xx2.reshape(ROWS, COLS))

    return lax.cond(overflow, slow, fast, (x2, idx, stats)) if _STAGE == 2 else ((flags, stats) if _STAGE == 0 else (flags, stats, idx, overflow))


def kernel(input):
    return _pipeline(input)


# speedup vs baseline: 1.8378x; 1.0022x over previous
import jax
import jax.numpy as jnp
from jax import lax
from jax.experimental import pallas as pl
from jax.experimental.pallas import tpu as pltpu
from jax.experimental.pallas import tpu_sc as plsc

ROWS = 128
COLS = 32768
LANES = 16
NSLICES = COLS // LANES
CHUNK = 128
NCHUNKS = COLS // CHUNK
SLICES_PER_CHUNK = CHUNK // LANES
LCAP = 48
NUM_CORES = 2
NUM_SUBCORES = 16
NWORKERS = NUM_CORES * NUM_SUBCORES
ROWS_PER_W = ROWS // NWORKERS
MARGIN = 3e-3
UNROLL = 8

_GATHER_DNUMS = lax.GatherDimensionNumbers(
    offset_dims=(), collapsed_slice_dims=(0,), start_index_map=(0,)
)


def _perm(v, idx):
    return lax.gather(
        v,
        idx[:, None],
        _GATHER_DNUMS,
        slice_sizes=(1,),
        mode=lax.GatherScatterMode.PROMISE_IN_BOUNDS,
    )


def _mk_helpers():
    lane = lax.iota(jnp.int32, LANES)
    bfly = [jnp.bitwise_xor(lane, sh) for sh in (1, 2, 4, 8)]

    def allmax(v):
        for idx in bfly:
            v = jnp.maximum(v, _perm(v, idx))
        return v

    def allsum(v):
        for idx in bfly:
            v = v + _perm(v, idx)
        return v

    return lane, allmax, allsum


_ONES = lambda: jnp.full((LANES,), 1.0, jnp.float32)
_ZERO = lambda: jnp.zeros((LANES,), jnp.float32)


def _body_a(x_hbm, flags_hbm, stats_hbm, rowa_v, rowb_v, flag_v, stats_v, sem_a, sem_b):
    cid = lax.axis_index("c")
    sid = lax.axis_index("s")
    wid = sid * NUM_CORES + cid
    r0 = wid * ROWS_PER_W

    lane, allmax, allsum = _mk_helpers()
    ones_v, zero_v = _ONES(), _ZERO()
    rowbuf = [rowa_v, rowb_v]
    sems = [sem_a, sem_b]

    d_in = [None] * ROWS_PER_W
    d_in[0] = pltpu.async_copy(x_hbm.at[r0], rowa_v, sem_a)

    for j in range(ROWS_PER_W):
        row_v = rowbuf[j % 2]
        d_in[j].wait()
        if j + 1 < ROWS_PER_W:
            d_in[j + 1] = pltpu.async_copy(
                x_hbm.at[r0 + j + 1], rowbuf[(j + 1) % 2], sems[(j + 1) % 2]
            )

        def maxbody(i, acc):
            for cu in range(2):
                g = i * 2 + cu
                base = g * CHUNK
                vs = [row_v[pl.ds(base + u * LANES, LANES)] for u in range(SLICES_PER_CHUNK)]
                m0 = jnp.maximum(vs[0], vs[1])
                m1 = jnp.maximum(vs[2], vs[3])
                m2 = jnp.maximum(vs[4], vs[5])
                m3 = jnp.maximum(vs[6], vs[7])
                mx = jnp.maximum(jnp.maximum(m0, m1), jnp.maximum(m2, m3))
                flag_v[pl.ds(g * LANES, LANES)] = mx
                acc = jnp.maximum(acc, mx)
            return acc

        acc = lax.fori_loop(0, NCHUNKS // 2, maxbody, row_v[pl.ds(0, LANES)])
        row_max = allmax(acc)

        def ks_at(t):
            def body(i, c):
                accs = list(c)
                base = i * (LANES * UNROLL)
                for u in range(UNROLL):
                    v = row_v[pl.ds(base + u * LANES, LANES)]
                    m = v > t
                    q = u % 4
                    accs[q] = accs[q] + jnp.where(m, ones_v, zero_v)
                    accs[4 + q] = accs[4 + q] + jnp.where(m, v, zero_v)
                return tuple(accs)

            accs = lax.fori_loop(0, NSLICES // UNROLL, body, (zero_v,) * 8)
            ka = (accs[0] + accs[1]) + (accs[2] + accs[3])
            sa = (accs[4] + accs[5]) + (accs[6] + accs[7])
            return allsum(ka), allsum(sa)

        lo = row_max - 1.001
        for _ in range(2):
            k, s = ks_at(lo)
            lo = (s - 1.0) / k
        lom = lo - MARGIN

        stats_v[pl.ds(0, LANES)] = lom
        stats_v[pl.ds(LANES, LANES)] = row_max
        pltpu.sync_copy(flag_v, flags_hbm.at[r0 + j])
        pltpu.sync_copy(stats_v, stats_hbm.at[r0 + j])


def _body_b(
    x2_hbm, idx_hbm, stats_hbm, out2_hbm,
    rowin0, rowin1, cand_a, cand_b, cand_c, cand_d, idx4_v, stats4_v,
    sem_g, sem_i0, sem_i1, sem_o0, sem_o1,
):
    cid = lax.axis_index("c")
    sid = lax.axis_index("s")
    wid = sid * NUM_CORES + cid
    r0 = wid * ROWS_PER_W

    lane, allmax, allsum = _mk_helpers()
    ones_v, zero_v = _ONES(), _ZERO()
    NR = ROWS_PER_W
    CU = 4
    cands = [cand_a, cand_b, cand_c, cand_d]
    rowin = [rowin0, rowin1]
    sem_in = [sem_i0, sem_i1]
    sem_out = [sem_o0, sem_o1]

    d_in = [None] * NR
    d_out = [None] * NR
    for j in range(2):
        d_in[j] = pltpu.async_copy(
            x2_hbm.at[pl.ds((r0 + j) * NCHUNKS, NCHUNKS)], rowin[j], sem_in[j]
        )
    pltpu.sync_copy(idx_hbm.at[pl.ds(r0, NR)], idx4_v)
    pltpu.sync_copy(stats_hbm.at[pl.ds(r0, NR)], stats4_v)
    d_g = [
        pltpu.async_copy(x2_hbm.at[idx4_v.at[j]], cands[j], sem_g)
        for j in range(NR)
    ]

    for j in range(NR):
        cand_v = cands[j]
        d_g[j].wait()
        lo = stats4_v[j, pl.ds(0, LANES)]
        row_max = stats4_v[j, pl.ds(LANES, LANES)]

        def ks_at(t):
            def body(i, c):
                accs = list(c)
                for cu in range(CU):
                    for u in range(SLICES_PER_CHUNK):
                        v = cand_v[i * CU + cu, pl.ds(u * LANES, LANES)]
                        m = v > t
                        q = u % 4
                        accs[q] = accs[q] + jnp.where(m, ones_v, zero_v)
                        accs[4 + q] = accs[4 + q] + jnp.where(m, v, zero_v)
                return tuple(accs)

            accs = lax.fori_loop(0, LCAP // CU, body, (zero_v,) * 8)
            ka = (accs[0] + accs[1]) + (accs[2] + accs[3])
            sa = (accs[4] + accs[5]) + (accs[6] + accs[7])
            return allsum(ka), allsum(sa)

        def fsum(t):
            def body(i, c):
                accs = list(c)
                for cu in range(CU):
                    for u in range(SLICES_PER_CHUNK):
                        v = cand_v[i * CU + cu, pl.ds(u * LANES, LANES)]
                        accs[u % 4] = accs[u % 4] + jnp.maximum(v - t, 0.0)
                return tuple(accs)

            accs = lax.fori_loop(0, LCAP // CU, body, (zero_v,) * 4)
            return allsum((accs[0] + accs[1]) + (accs[2] + accs[3]))

        for _ in range(3):
            k, s = ks_at(lo)
            lo = (s - 1.0) / k

        f_lo = fsum(lo)
        hi = jnp.minimum(lo + (f_lo - 1.0), row_max)
        hi = jnp.maximum(hi, lo)

        def bis(i, c):
            blo, bhi = c
            mid = 0.5 * (blo + bhi)
            gt = fsum(mid) > 1.0
            return (jnp.where(gt, mid, blo), jnp.where(gt, bhi, mid))

        lo, _ = lax.fori_loop(0, 12, bis, (lo, hi))

        k, s = ks_at(lo)
        tau = (s - 1.0) / k

        if j >= 1 and j + 1 < NR:
            d_out[j - 1].wait()
            d_in[j + 1] = pltpu.async_copy(
                x2_hbm.at[pl.ds((r0 + j + 1) * NCHUNKS, NCHUNKS)],
                rowin[(j + 1) % 2],
                sem_in[(j + 1) % 2],
            )

        d_in[j].wait()
        buf = rowin[j % 2]

        def outbody(i, c):
            for cu in range(2):
                for u in range(SLICES_PER_CHUNK):
                    sl = (i * 2 + cu, pl.ds(u * LANES, LANES))
                    buf[sl] = jnp.maximum(buf[sl] - tau, 0.0)
            return c

        lax.fori_loop(0, NCHUNKS // 2, outbody, 0)
        d_out[j] = pltpu.async_copy(
            buf, out2_hbm.at[pl.ds((r0 + j) * NCHUNKS, NCHUNKS)], sem_out[j % 2]
        )

    d_out[NR - 2].wait()
    d_out[NR - 1].wait()


def _body_c(x_hbm, out_hbm, row_v):
    cid = lax.axis_index("c")
    sid = lax.axis_index("s")
    wid = sid * NUM_CORES + cid

    lane, allmax, allsum = _mk_helpers()
    ones_v, zero_v = _ONES(), _ZERO()

    def do_row(j, carry):
        r = wid * ROWS_PER_W + j
        pltpu.sync_copy(x_hbm.at[r], row_v)

        def maxbody(i, acc):
            base = i * (LANES * UNROLL)
            for u in range(UNROLL):
                acc = jnp.maximum(acc, row_v[pl.ds(base + u * LANES, LANES)])
            return acc

        acc = lax.fori_loop(0, NSLICES // UNROLL, maxbody, row_v[pl.ds(0, LANES)])
        row_max = allmax(acc)

        def ks_at(t):
            def body(i, c):
                accs = list(c)
                base = i * (LANES * UNROLL)
                for u in range(UNROLL):
                    v = row_v[pl.ds(base + u * LANES, LANES)]
                    m = v > t
                    j = u % 4
                    accs[j] = accs[j] + jnp.where(m, ones_v, zero_v)
                    accs[4 + j] = accs[4 + j] + jnp.where(m, v, zero_v)
                return tuple(accs)

            accs = lax.fori_loop(0, NSLICES // UNROLL, body, (zero_v,) * 8)
            ka = (accs[0] + accs[1]) + (accs[2] + accs[3])
            sa = (accs[4] + accs[5]) + (accs[6] + accs[7])
            return allsum(ka), allsum(sa)

        def fsum(tau):
            def body(i, c):
                accs = list(c)
                base = i * (LANES * UNROLL)
                for u in range(UNROLL):
                    v = row_v[pl.ds(base + u * LANES, LANES)]
                    accs[u % 4] = accs[u % 4] + jnp.maximum(v - tau, 0.0)
                return tuple(accs)

            accs = lax.fori_loop(0, NSLICES // UNROLL, body, (zero_v,) * 4)
            return allsum((accs[0] + accs[1]) + (accs[2] + accs[3]))

        lo = row_max - 1.001
        for _ in range(4):
            k, s = ks_at(lo)
            lo = (s - 1.0) / k

        f_lo = fsum(lo)
        hi = jnp.minimum(lo + (f_lo - 1.0), row_max)
        hi = jnp.maximum(hi, lo)

        def bis(i, c):
            blo, bhi = c
            mid = 0.5 * (blo + bhi)
            gt = fsum(mid) > 1.0
            return (jnp.where(gt, mid, blo), jnp.where(gt, bhi, mid))

        lo, _ = lax.fori_loop(0, 26, bis, (lo, hi))

        k, s = ks_at(lo)
        tau = (s - 1.0) / k

        def outbody(i, c):
            base = i * (LANES * UNROLL)
            for u in range(UNROLL):
                sl = pl.ds(base + u * LANES, LANES)
                row_v[sl] = jnp.maximum(row_v[sl] - tau, 0.0)
            return c

        lax.fori_loop(0, NSLICES // UNROLL, outbody, 0)
        pltpu.sync_copy(row_v, out_hbm.at[r])
        return carry

    lax.fori_loop(0, ROWS_PER_W, do_row, 0)


def _mesh():
    return plsc.VectorSubcoreMesh(core_axis_name="c", subcore_axis_name="s")


def _kernel_a(x):
    fn = pl.kernel(
        _body_a,
        out_type=(
            jax.ShapeDtypeStruct((ROWS, NCHUNKS * LANES), jnp.float32),
            jax.ShapeDtypeStruct((ROWS, 2 * LANES), jnp.float32),
        ),
        mesh=_mesh(),
        scratch_types=[
            pltpu.VMEM((COLS,), jnp.float32),
            pltpu.VMEM((COLS,), jnp.float32),
            pltpu.VMEM((NCHUNKS * LANES,), jnp.float32),
            pltpu.VMEM((2 * LANES,), jnp.float32),
            pltpu.SemaphoreType.DMA,
            pltpu.SemaphoreType.DMA,
        ],
    )
    return fn(x)


def _kernel_b(x2, idx, stats):
    fn = pl.kernel(
        _body_b,
        out_type=jax.ShapeDtypeStruct((ROWS * NCHUNKS, CHUNK), jnp.float32),
        mesh=_mesh(),
        scratch_types=[
            pltpu.VMEM((NCHUNKS, CHUNK), jnp.float32),
            pltpu.VMEM((NCHUNKS, CHUNK), jnp.float32),
            pltpu.VMEM((LCAP, CHUNK), jnp.float32),
            pltpu.VMEM((LCAP, CHUNK), jnp.float32),
            pltpu.VMEM((LCAP, CHUNK), jnp.float32),
            pltpu.VMEM((LCAP, CHUNK), jnp.float32),
            pltpu.VMEM((ROWS_PER_W, LCAP), jnp.int32),
            pltpu.VMEM((ROWS_PER_W, 2 * LANES), jnp.float32),
            pltpu.SemaphoreType.DMA,
            pltpu.SemaphoreType.DMA,
            pltpu.SemaphoreType.DMA,
            pltpu.SemaphoreType.DMA,
            pltpu.SemaphoreType.DMA,
        ],
    )
    return fn(x2, idx, stats)


def _kernel_c(x):
    fn = pl.kernel(
        _body_c,
        out_type=jax.ShapeDtypeStruct((ROWS, COLS), jnp.float32),
        mesh=_mesh(),
        scratch_types=[pltpu.VMEM((COLS,), jnp.float32)],
    )
    return fn(x)


@jax.jit
def _pipeline(x):
    flags, stats = _kernel_a(x)
    score = flags.reshape(ROWS, NCHUNKS, LANES).max(axis=2)
    lom = stats[:, 0:1]
    flag = score > lom
    cnt = jnp.sum(flag.astype(jnp.int32), axis=1)
    overflow = jnp.any(cnt > LCAP)

    _, top_idx = lax.top_k(score, LCAP)
    base = (jnp.arange(ROWS, dtype=jnp.int32) * NCHUNKS)[:, None]
    idx = base + top_idx.astype(jnp.int32)

    x2 = x.reshape(ROWS * NCHUNKS, CHUNK)

    def fast(ops):
        xx2, iidx, sstats = ops
        out2 = _kernel_b(xx2, iidx, sstats)
        return out2.reshape(ROWS, COLS)

    def slow(ops):
        xx2, _, _ = ops
        return _kernel_c(xx2.reshape(ROWS, COLS))

    return lax.cond(overflow, slow, fast, (x2, idx, stats))


def kernel(input):
    return _pipeline(input)
